# baseline (speedup 1.0000x reference)
import functools

import jax
import jax.numpy as jnp
from jax import lax
from jax.experimental import pallas as pl
from jax.experimental.pallas import tpu as pltpu
from jax.experimental.pallas import tpu_sc as plsc

D_MODEL = 64
IDX_W = 128
LANES = 16


@functools.lru_cache(maxsize=None)
def _make_gather(ntot: int):
    info = plsc.get_sparse_core_info()
    nc, ns = info.num_cores, info.num_subcores
    nw = nc * ns
    per_w = ntot // nw
    assert per_w * nw == ntot and per_w % IDX_W == 0
    nh = per_w // IDX_W
    rows_w = per_w // IDX_W
    out_rows_w = per_w // 2
    mesh = plsc.VectorSubcoreMesh(core_axis_name="c", subcore_axis_name="s")

    @functools.partial(
        pl.kernel,
        out_type=jax.ShapeDtypeStruct((ntot // 2, 128), jnp.float32),
        mesh=mesh,
        compiler_params=pltpu.CompilerParams(use_tc_tiling_on_sc=False,
                                             needs_layout_passes=False),
        scratch_types=[
            pltpu.VMEM((rows_w, IDX_W), jnp.int32),
            pltpu.VMEM((2, IDX_W), jnp.int32),
            pltpu.VMEM((2 * IDX_W, 128), jnp.float32),
            pltpu.VMEM((IDX_W, 128), jnp.float32),
            pltpu.SemaphoreType.DMA,
            pltpu.SemaphoreType.DMA,
            pltpu.SemaphoreType.DMA,
            pltpu.SemaphoreType.DMA,
        ],
    )
    def gather_k(idx_hbm, table_hbm, out_hbm, x_all, xp_v, rows_v, out_v,
                 gsem0, gsem1, ssem0, ssem1):
        wid = lax.axis_index("s") * nc + lax.axis_index("c")
        base_row = wid * rows_w
        base_out = wid * out_rows_w
        gsems = (gsem0, gsem1)
        ssems = (ssem0, ssem1)
        parity_lane = lax.rem(lax.iota(jnp.int32, LANES), 2)

        def compute_xp(h):
            slot = lax.rem(h, 2)
            for c in range(IDX_W // LANES):
                xv = x_all[h, pl.ds(c * LANES, LANES)]
                xp_v[slot, pl.ds(c * LANES, LANES)] = (
                    lax.shift_right_logical(xv, 1))

        def store_desc(h, p):
            return pltpu.make_async_copy(
                out_v.at[pl.ds(p * (IDX_W // 2), IDX_W // 2)],
                out_hbm.at[pl.ds(base_out + h * (IDX_W // 2),
                                 IDX_W // 2)], ssems[p])

        def select(h, p):
            def jg_body(jg, carry):
                j0 = jg * LANES
                jvec = j0 + lax.iota(jnp.int32, LANES)
                hvec = lax.rem(x_all[h, pl.ds(j0, LANES)], 2)
                col_base = hvec * D_MODEL
                rvec = lax.shift_right_logical(jvec, 1) + p * (IDX_W // 2)
                gvec = jvec + p * IDX_W
                cvec_base = parity_lane * D_MODEL
                for dd in range(D_MODEL):
                    vals = plsc.load_gather(
                        rows_v, [gvec, col_base + dd])
                    plsc.store_scatter(
                        out_v, [rvec, cvec_base + dd], vals)
                return carry
            lax.fori_loop(0, IDX_W // LANES, jg_body, 0, unroll=False)

        pltpu.sync_copy(idx_hbm.at[pl.ds(base_row, rows_w)], x_all)
        compute_xp(0)
        pltpu.async_copy(table_hbm.at[xp_v.at[0]],
                         rows_v.at[pl.ds(0, IDX_W)], gsem0)

        def two_halves(b, carry):
            for q in (0, 1):
                h = b * 2 + q
                hn = h + 1
                hn_c = lax.min(hn, nh - 1)
                compute_xp(hn_c)
                pltpu.async_copy(
                    table_hbm.at[xp_v.at[lax.rem(hn_c, 2)]],
                    rows_v.at[pl.ds(((q + 1) % 2) * IDX_W, IDX_W)],
                    gsems[(q + 1) % 2])
                pltpu.make_async_copy(
                    table_hbm.at[xp_v.at[lax.rem(h, 2)]],
                    rows_v.at[pl.ds(q * IDX_W, IDX_W)], gsems[q]).wait()
                store_desc(h, q).wait()
                select(h, q)
                store_desc(h, q).start()
            return carry

        for h in (0, 1):
            q = h % 2
            hn = h + 1
            compute_xp(hn)
            pltpu.async_copy(table_hbm.at[xp_v.at[hn % 2]],
                             rows_v.at[pl.ds((hn % 2) * IDX_W, IDX_W)],
                             gsems[hn % 2])
            pltpu.make_async_copy(table_hbm.at[xp_v.at[q]],
                                  rows_v.at[pl.ds(q * IDX_W, IDX_W)],
                                  gsems[q]).wait()
            select(h, q)
            store_desc(h, q).start()

        lax.fori_loop(1, nh // 2, two_halves, 0, unroll=False)

        pltpu.make_async_copy(table_hbm.at[xp_v.at[0]],
                              rows_v.at[pl.ds(0, IDX_W)], gsems[0]).wait()
        store_desc(nh - 2, 0).wait()
        store_desc(nh - 1, 1).wait()

    return gather_k


def _mm_body(c_ref, w_ref, o_ref):
    o_ref[...] = lax.dot_general(
        c_ref[...], w_ref[...],
        dimension_numbers=(((1,), (1,)), ((), ())),
        preferred_element_type=jnp.float32,
    )


def _cond_proj(condition, w_cond):
    b = condition.shape[0]
    return pl.pallas_call(
        _mm_body,
        out_shape=jax.ShapeDtypeStruct((b, w_cond.shape[0]), jnp.float32),
    )(condition, w_cond)


def kernel(x, condition, table, W_cond):
    b, l = x.shape
    ntot = b * l
    idx = x.reshape(ntot // IDX_W, IDX_W).astype(jnp.int32)
    table2 = table.reshape(table.shape[0] // 2, 2 * D_MODEL)
    gather_k = _make_gather(ntot)
    out2 = gather_k(idx, table2)
    x_emb = out2.reshape(b, l, D_MODEL)
    cond_emb = _cond_proj(condition, W_cond)
    return (x_emb, cond_emb)


# speedup vs baseline: 2.3966x; 2.3966x over previous
import functools

import jax
import jax.numpy as jnp
from jax import lax
from jax.experimental import pallas as pl
from jax.experimental.pallas import tpu as pltpu
from jax.experimental.pallas import tpu_sc as plsc

D_MODEL = 64
IDX_W = 128
BLK = 1024
HALF = BLK // 2
KH = HALF // IDX_W


@functools.lru_cache(maxsize=None)
def _make_gather(ntot: int):
    info = plsc.get_sparse_core_info()
    nc, ns = info.num_cores, info.num_subcores
    nw = nc * ns
    per_w = ntot // nw
    assert per_w * nw == ntot and per_w % BLK == 0
    nblk = per_w // BLK
    rows_per_blk = BLK // IDX_W
    n_idx_rows = ntot // IDX_W
    mesh = plsc.VectorSubcoreMesh(core_axis_name="c", subcore_axis_name="s")

    @functools.partial(
        pl.kernel,
        out_type=jax.ShapeDtypeStruct((ntot, D_MODEL), jnp.float32),
        mesh=mesh,
        compiler_params=pltpu.CompilerParams(use_tc_tiling_on_sc=False),
        scratch_types=[
            pltpu.VMEM((2, rows_per_blk, IDX_W), jnp.int32),
            pltpu.VMEM((2, HALF, D_MODEL), jnp.float32),
            pltpu.SemaphoreType.DMA,
            pltpu.SemaphoreType.DMA,
            pltpu.SemaphoreType.DMA,
            pltpu.SemaphoreType.DMA,
        ],
    )
    def gather_k(idx_hbm, table_hbm, out_hbm, idx_v, rows_v, gsem, ssem0,
                 ssem1, isem):
        wid = lax.axis_index("s") * nc + lax.axis_index("c")
        base = wid * per_w
        base_row = wid * (per_w // IDX_W)
        ssems = (ssem0, ssem1)

        def idx_fetch(b):
            row = lax.min(base_row + b * rows_per_blk,
                          n_idx_rows - rows_per_blk)
            row = pl.multiple_of(row, 8)
            return pltpu.make_async_copy(
                idx_hbm.at[pl.ds(row, rows_per_blk)], idx_v.at[b % 2], isem)

        def store_desc(p, off):
            return pltpu.make_async_copy(
                rows_v.at[p], out_hbm.at[pl.ds(off, HALF)], ssems[p])

        def half_iter(b, half, drain):
            p = half
            off = base + b * BLK + half * HALF
            if drain:
                store_desc(p, off).wait()
            copies = [
                pltpu.async_copy(
                    table_hbm.at[idx_v.at[b % 2].at[half * KH + jj]],
                    rows_v.at[p].at[pl.ds(jj * IDX_W, IDX_W)],
                    gsem)
                for jj in range(KH)
            ]
            if half == 0:
                idx_fetch(b + 1).start()
            for c in copies:
                c.wait()
            store_desc(p, off).start()

        idx_fetch(0).start()
        idx_fetch(0).wait()
        half_iter(0, 0, drain=False)
        half_iter(0, 1, drain=False)

        def body(b, carry):
            idx_fetch(b).wait()
            half_iter(b, 0, drain=True)
            half_iter(b, 1, drain=True)
            return carry

        lax.fori_loop(1, nblk, body, 0, unroll=False)

        idx_fetch(nblk).wait()
        store_desc(0, base + (nblk - 1) * BLK).wait()
        store_desc(1, base + (nblk - 1) * BLK + HALF).wait()

    return gather_k


def _mm_body(c_ref, w_ref, o_ref):
    o_ref[...] = lax.dot_general(
        c_ref[...], w_ref[...],
        dimension_numbers=(((1,), (1,)), ((), ())),
        preferred_element_type=jnp.float32,
    )


def _cond_proj(condition, w_cond):
    b = condition.shape[0]
    return pl.pallas_call(
        _mm_body,
        out_shape=jax.ShapeDtypeStruct((b, w_cond.shape[0]), jnp.float32),
    )(condition, w_cond)


def kernel(x, condition, table, W_cond):
    b, l = x.shape
    ntot = b * l
    idx = x.reshape(ntot // IDX_W, IDX_W).astype(jnp.int32)
    gather_k = _make_gather(ntot)
    x_emb = gather_k(idx, table).reshape(b, l, D_MODEL)
    cond_emb = _cond_proj(condition, W_cond)
    return (x_emb, cond_emb)
